# baseline (speedup 1.0000x reference)
import jax
import jax.numpy as jnp
from jax.experimental import pallas as pl
from jax.experimental.pallas import tpu as pltpu

_T = 100
_TBLK = 5


def _body(xt_ref, d_ref, o_ref, st_ref):
    i = pl.program_id(0)

    @pl.when(i == 0)
    def _():
        st = ((1.0 - xt_ref[...] * d_ref[...]) * (_T - 1)).astype(jnp.int32)
        st_ref[...] = jnp.clip(st, 0, _T - 1)

    f, b = st_ref.shape
    t = jax.lax.broadcasted_iota(jnp.int32, (_TBLK, f, b), 0) + i * _TBLK
    o_ref[...] = (t == st_ref[...][None, :, :]).astype(jnp.float32)


def kernel(x, delays):
    b, f = x.shape
    out_tfb = pl.pallas_call(
        _body,
        grid=(_T // _TBLK,),
        in_specs=[
            pl.BlockSpec((f, b), lambda i: (0, 0)),
            pl.BlockSpec((f, 1), lambda i: (0, 0)),
        ],
        out_specs=pl.BlockSpec((_TBLK, f, b), lambda i: (i, 0, 0)),
        out_shape=jax.ShapeDtypeStruct((_T, f, b), jnp.float32),
        scratch_shapes=[pltpu.VMEM((f, b), jnp.int32)],
    )(x.T, delays[:, None])
    return jnp.transpose(out_tfb, (2, 0, 1))


# speedup vs baseline: 1.0221x; 1.0221x over previous
import jax
import jax.numpy as jnp
from jax.experimental import pallas as pl
from jax.experimental.pallas import tpu as pltpu

_T = 100
_TBLK = 1


def _body(xt_ref, d_ref, o_ref, st_ref):
    i = pl.program_id(0)

    @pl.when(i == 0)
    def _():
        st = ((1.0 - xt_ref[...] * d_ref[...]) * (_T - 1)).astype(jnp.int32)
        st_ref[...] = jnp.clip(st, 0, _T - 1)

    f, b = st_ref.shape
    t = jax.lax.broadcasted_iota(jnp.int32, (_TBLK, f, b), 0) + i * _TBLK
    o_ref[...] = (t == st_ref[...][None, :, :]).astype(jnp.float32)


def kernel(x, delays):
    b, f = x.shape
    out_tfb = pl.pallas_call(
        _body,
        grid=(_T // _TBLK,),
        in_specs=[
            pl.BlockSpec((f, b), lambda i: (0, 0)),
            pl.BlockSpec((f, 1), lambda i: (0, 0)),
        ],
        out_specs=pl.BlockSpec((_TBLK, f, b), lambda i: (i, 0, 0)),
        out_shape=jax.ShapeDtypeStruct((_T, f, b), jnp.float32),
        scratch_shapes=[pltpu.VMEM((f, b), jnp.int32)],
    )(x.T, delays[:, None])
    return jnp.transpose(out_tfb, (2, 0, 1))
